# Initial kernel scaffold; baseline (speedup 1.0000x reference)
import jax
import jax.numpy as jnp
from jax.experimental import pallas as pl


def kernel(x, edge_index, W0, b0, W1, b1, W2, b2):
    raise NotImplementedError("write your pallas kernel here")


# speedup vs baseline: 5.2019x; 5.2019x over previous
import jax
import jax.numpy as jnp
from jax import lax
from jax.experimental import pallas as pl
from jax.experimental.pallas import tpu as pltpu
from jax.experimental.pallas import tpu_sc as plsc

N = 10000
E = 160000
D = 256
HALF = 128
NPAD = 10240
NC, NS = 2, 16
E_PAD = 163840
EPT = E_PAD // NS
BLK = 128
NBLK = EPT // BLK
CHUNK = 8
ROWS_OUT = N // NS
ZROWS = NPAD // NS
HB = 16

def _sc_scatter_body(g_hbm, ridx_hbm, cidx_hbm, zero_hbm, out_hbm,
                     ridx_v, cidx_v, rows_v, acc_sh, sem):
    c = lax.axis_index("c")
    s = lax.axis_index("s")
    pltpu.sync_copy(zero_hbm, acc_sh.at[pl.ds(s * ZROWS, ZROWS), :])
    plsc.subcore_barrier()

    base = (c * E_PAD + s * EPT) // BLK

    def _block(t, carry):
        rb = pl.multiple_of(base + t * CHUNK, CHUNK)
        pltpu.sync_copy(ridx_hbm.at[pl.ds(rb, CHUNK), :], ridx_v)
        pltpu.sync_copy(cidx_hbm.at[pl.ds(rb, CHUNK), :], cidx_v)
        for j in range(CHUNK):
            pltpu.async_copy(g_hbm.at[ridx_v.at[j]], rows_v, sem).wait()
            pltpu.sync_copy(rows_v, acc_sh.at[cidx_v.at[j]], add=True)
        return carry

    lax.fori_loop(0, NBLK // CHUNK, _block, 0)
    plsc.subcore_barrier()
    pltpu.sync_copy(acc_sh.at[pl.ds(s * ZROWS, ZROWS), :],
                    out_hbm.at[c, pl.ds(s * ZROWS, ZROWS), :])


import functools


@functools.lru_cache(maxsize=None)
def _sc_kernels():
    mesh = plsc.VectorSubcoreMesh(core_axis_name="c", subcore_axis_name="s",
                                  num_cores=NC, num_subcores=NS)
    scatter = pl.kernel(
        _sc_scatter_body,
        out_type=jax.ShapeDtypeStruct((NC, NPAD, HALF), jnp.float32),
        mesh=mesh,
        scratch_types=[
            pltpu.VMEM((CHUNK, BLK), jnp.int32),
            pltpu.VMEM((CHUNK, BLK), jnp.int32),
            pltpu.VMEM((BLK, HALF), jnp.float32),
            pltpu.VMEM_SHARED((NPAD, HALF), jnp.float32),
            pltpu.SemaphoreType.DMA,
        ],
    )
    return scatter


RB = 1000
GRID = N // RB
_DOTDIMS = (((1,), (1,)), ((), ()))


def _matmul(xin, w_ref, b_ref):
    out = lax.dot_general(xin, w_ref[...], _DOTDIMS,
                          preferred_element_type=jnp.float32,
                          precision=lax.Precision.HIGHEST)
    return jnp.maximum(out + b_ref[...], 0.0)


def _tc_first_body(deg_ref, x_ref, w_ref, b_ref, h_ref, g_ref, dis_ref):
    dis = lax.rsqrt(deg_ref[...] + 1.0)
    dis_ref[...] = dis
    h = _matmul(x_ref[...], w_ref, b_ref)
    h_ref[...] = h
    g = dis * h
    g_ref[0, :, :] = g[:, :HALF]
    g_ref[1, :, :] = g[:, HALF:]


def _tc_mid_body(acc_ref, hp_ref, dis_ref, w_ref, b_ref, h_ref, g_ref):
    dis = dis_ref[...]
    xin = (jnp.concatenate([dis * acc_ref[0], dis * acc_ref[1]], axis=1)
           + (dis * dis) * hp_ref[...])
    h = _matmul(xin, w_ref, b_ref)
    h_ref[...] = h
    g = dis * h
    g_ref[0, :, :] = g[:, :HALF]
    g_ref[1, :, :] = g[:, HALF:]


def _tc_final_body(acc_ref, hp_ref, dis_ref, out_ref):
    dis = dis_ref[...]
    out_ref[...] = (jnp.concatenate([dis * acc_ref[0], dis * acc_ref[1]], axis=1)
                    + (dis * dis) * hp_ref[...])


_spec_rows = pl.BlockSpec((RB, D), lambda i: (i, 0))
_spec_acc = pl.BlockSpec((2, RB, HALF), lambda i: (0, i, 0))
_spec_w = pl.BlockSpec((D, D), lambda i: (0, 0))
_spec_b = pl.BlockSpec((1, D), lambda i: (0, 0))
_spec_col = pl.BlockSpec((RB, 1), lambda i: (i, 0))

_tc_first = pl.pallas_call(
    _tc_first_body,
    grid=(GRID,),
    in_specs=[_spec_col, _spec_rows, _spec_w, _spec_b],
    out_specs=[_spec_rows, _spec_acc, _spec_col],
    out_shape=[
        jax.ShapeDtypeStruct((N, D), jnp.float32),
        jax.ShapeDtypeStruct((2, N, HALF), jnp.float32),
        jax.ShapeDtypeStruct((N, 1), jnp.float32),
    ],
)

_tc_mid = pl.pallas_call(
    _tc_mid_body,
    grid=(GRID,),
    in_specs=[_spec_acc, _spec_rows, _spec_col, _spec_w, _spec_b],
    out_specs=[_spec_rows, _spec_acc],
    out_shape=[
        jax.ShapeDtypeStruct((N, D), jnp.float32),
        jax.ShapeDtypeStruct((2, N, HALF), jnp.float32),
    ],
)

_tc_final = pl.pallas_call(
    _tc_final_body,
    grid=(GRID,),
    in_specs=[_spec_acc, _spec_rows, _spec_col],
    out_specs=_spec_rows,
    out_shape=jax.ShapeDtypeStruct((N, D), jnp.float32),
)


def kernel(x, edge_index, W0, b0, W1, b1, W2, b2):
    ei = edge_index.astype(jnp.int32)
    row, col = ei[0], ei[1]
    pad = E_PAD - E
    row_p = jnp.concatenate([row, jnp.zeros((pad,), jnp.int32)])
    col_p = jnp.concatenate([col, jnp.full((pad,), N, jnp.int32)])
    ridx = jnp.concatenate([row_p, row_p + N]).reshape(-1, BLK)
    cidx = jnp.concatenate([col_p, col_p]).reshape(-1, BLK)

    _sc_scatter = _sc_kernels()
    zrows = jnp.zeros((ZROWS, HALF), jnp.float32)
    ones_g = jnp.ones((2 * N, HALF), jnp.float32)
    hist = _sc_scatter(ones_g, ridx, cidx, zrows)
    deg = hist[0, :N, 0][:, None]

    b0r, b1r, b2r = b0[None, :], b1[None, :], b2[None, :]
    h1, g1, dis = _tc_first(deg, x, W0, b0r)
    acc = _sc_scatter(g1.reshape(2 * N, HALF), ridx, cidx, zrows)
    h2, g2 = _tc_mid(acc, h1, dis, W1, b1r)
    acc = _sc_scatter(g2.reshape(2 * N, HALF), ridx, cidx, zrows)
    h3, g3 = _tc_mid(acc, h2, dis, W2, b2r)
    acc = _sc_scatter(g3.reshape(2 * N, HALF), ridx, cidx, zrows)
    return _tc_final(acc, h3, dis)


# speedup vs baseline: 7.7047x; 1.4811x over previous
import jax
import jax.numpy as jnp
from jax import lax
from jax.experimental import pallas as pl
from jax.experimental.pallas import tpu as pltpu
from jax.experimental.pallas import tpu_sc as plsc

N = 10000
E = 160000
D = 256
HALF = 128
NPAD = 10240
NC, NS = 2, 16
E_PAD = 163840
EPT = E_PAD // NS
BLK = 128
NBLK = EPT // BLK
CHUNK = 16
ROWS_OUT = N // NS
ZROWS = NPAD // NS
HB = 128

def _sc_scatter_body(g_hbm, ridx_hbm, cidx_hbm, zero_hbm, out_hbm,
                     ridx_v, cidx_v, rows0_v, rows1_v, acc_sh, sem0, sem1):
    c = lax.axis_index("c")
    s = lax.axis_index("s")
    pltpu.sync_copy(zero_hbm, acc_sh.at[pl.ds(s * ZROWS, ZROWS), :])
    plsc.subcore_barrier()

    base = (c * E_PAD + s * EPT) // BLK
    bufs = (rows0_v, rows1_v)
    sems = (sem0, sem1)

    def _block(t, carry):
        rb = pl.multiple_of(base + t * CHUNK, CHUNK)
        pltpu.sync_copy(ridx_hbm.at[pl.ds(rb, CHUNK), :], ridx_v)
        pltpu.sync_copy(cidx_hbm.at[pl.ds(rb, CHUNK), :], cidx_v)
        pltpu.async_copy(g_hbm.at[ridx_v.at[0]], bufs[0], sems[0])
        for j in range(CHUNK):
            b = j & 1
            if j + 1 < CHUNK:
                pltpu.async_copy(g_hbm.at[ridx_v.at[j + 1]], bufs[1 - b],
                                 sems[1 - b])
            pltpu.make_async_copy(g_hbm.at[ridx_v.at[j]], bufs[b],
                                  sems[b]).wait()
            pltpu.sync_copy(bufs[b], acc_sh.at[cidx_v.at[j]], add=True)
        return carry

    lax.fori_loop(0, NBLK // CHUNK, _block, 0)
    plsc.subcore_barrier()
    pltpu.sync_copy(acc_sh.at[pl.ds(s * ZROWS, ZROWS), :],
                    out_hbm.at[c, pl.ds(s * ZROWS, ZROWS), :])


def _sc_hist_body(ones_hbm, cidx_hbm, zero_hbm, out_hbm,
                  cidx_v, ones_v, acc_sh):
    c = lax.axis_index("c")
    s = lax.axis_index("s")
    pltpu.sync_copy(zero_hbm, acc_sh.at[pl.ds(s * ZROWS, ZROWS), :])
    pltpu.sync_copy(ones_hbm, ones_v)
    plsc.subcore_barrier()

    base = (c * E_PAD + s * EPT) // BLK

    def _block(t, carry):
        rb = pl.multiple_of(base + t * CHUNK, CHUNK)
        pltpu.sync_copy(cidx_hbm.at[pl.ds(rb, CHUNK), :], cidx_v)
        for j in range(CHUNK):
            pltpu.sync_copy(ones_v, acc_sh.at[cidx_v.at[j]], add=True)
        return carry

    lax.fori_loop(0, NBLK // CHUNK, _block, 0)
    plsc.subcore_barrier()
    pltpu.sync_copy(acc_sh.at[pl.ds(s * ZROWS, ZROWS), :],
                    out_hbm.at[c, pl.ds(s * ZROWS, ZROWS), :])


import functools


@functools.lru_cache(maxsize=None)
def _sc_kernels():
    mesh = plsc.VectorSubcoreMesh(core_axis_name="c", subcore_axis_name="s",
                                  num_cores=NC, num_subcores=NS)
    scatter = pl.kernel(
        _sc_scatter_body,
        out_type=jax.ShapeDtypeStruct((NC, NPAD, HALF), jnp.float32),
        mesh=mesh,
        scratch_types=[
            pltpu.VMEM((CHUNK, BLK), jnp.int32),
            pltpu.VMEM((CHUNK, BLK), jnp.int32),
            pltpu.VMEM((BLK, HALF), jnp.float32),
            pltpu.VMEM((BLK, HALF), jnp.float32),
            pltpu.VMEM_SHARED((NPAD, HALF), jnp.float32),
            pltpu.SemaphoreType.DMA,
            pltpu.SemaphoreType.DMA,
        ],
    )
    hist = pl.kernel(
        _sc_hist_body,
        out_type=jax.ShapeDtypeStruct((NC, NPAD, HB), jnp.float32),
        mesh=mesh,
        scratch_types=[
            pltpu.VMEM((CHUNK, BLK), jnp.int32),
            pltpu.VMEM((BLK, HB), jnp.float32),
            pltpu.VMEM_SHARED((NPAD, HB), jnp.float32),
        ],
    )
    return scatter, hist


RB = 1000
GRID = N // RB
_DOTDIMS = (((1,), (1,)), ((), ()))


def _matmul(xin, w_ref, b_ref):
    out = lax.dot_general(xin, w_ref[...], _DOTDIMS,
                          preferred_element_type=jnp.float32,
                          precision=lax.Precision.HIGHEST)
    return jnp.maximum(out + b_ref[...], 0.0)


def _tc_first_body(deg_ref, x_ref, w_ref, b_ref, h_ref, g_ref, dis_ref):
    dis = lax.rsqrt(deg_ref[...] + 1.0)
    dis_ref[...] = dis
    h = _matmul(x_ref[...], w_ref, b_ref)
    h_ref[...] = h
    g = dis * h
    g_ref[0, :, :] = g[:, :HALF]
    g_ref[1, :, :] = g[:, HALF:]


def _tc_mid_body(acc_ref, hp_ref, dis_ref, w_ref, b_ref, h_ref, g_ref):
    dis = dis_ref[...]
    xin = (jnp.concatenate([dis * acc_ref[0], dis * acc_ref[1]], axis=1)
           + (dis * dis) * hp_ref[...])
    h = _matmul(xin, w_ref, b_ref)
    h_ref[...] = h
    g = dis * h
    g_ref[0, :, :] = g[:, :HALF]
    g_ref[1, :, :] = g[:, HALF:]


def _tc_final_body(acc_ref, hp_ref, dis_ref, out_ref):
    dis = dis_ref[...]
    out_ref[...] = (jnp.concatenate([dis * acc_ref[0], dis * acc_ref[1]], axis=1)
                    + (dis * dis) * hp_ref[...])


_spec_rows = pl.BlockSpec((RB, D), lambda i: (i, 0))
_spec_acc = pl.BlockSpec((2, RB, HALF), lambda i: (0, i, 0))
_spec_w = pl.BlockSpec((D, D), lambda i: (0, 0))
_spec_b = pl.BlockSpec((1, D), lambda i: (0, 0))
_spec_col = pl.BlockSpec((RB, 1), lambda i: (i, 0))

_tc_first = pl.pallas_call(
    _tc_first_body,
    grid=(GRID,),
    in_specs=[_spec_col, _spec_rows, _spec_w, _spec_b],
    out_specs=[_spec_rows, _spec_acc, _spec_col],
    out_shape=[
        jax.ShapeDtypeStruct((N, D), jnp.float32),
        jax.ShapeDtypeStruct((2, N, HALF), jnp.float32),
        jax.ShapeDtypeStruct((N, 1), jnp.float32),
    ],
)

_tc_mid = pl.pallas_call(
    _tc_mid_body,
    grid=(GRID,),
    in_specs=[_spec_acc, _spec_rows, _spec_col, _spec_w, _spec_b],
    out_specs=[_spec_rows, _spec_acc],
    out_shape=[
        jax.ShapeDtypeStruct((N, D), jnp.float32),
        jax.ShapeDtypeStruct((2, N, HALF), jnp.float32),
    ],
)

_tc_final = pl.pallas_call(
    _tc_final_body,
    grid=(GRID,),
    in_specs=[_spec_acc, _spec_rows, _spec_col],
    out_specs=_spec_rows,
    out_shape=jax.ShapeDtypeStruct((N, D), jnp.float32),
)


def kernel(x, edge_index, W0, b0, W1, b1, W2, b2):
    ei = edge_index.astype(jnp.int32)
    row, col = ei[0], ei[1]
    pad = E_PAD - E
    row_p = jnp.concatenate([row, jnp.zeros((pad,), jnp.int32)])
    col_p = jnp.concatenate([col, jnp.full((pad,), N, jnp.int32)])
    ridx = jnp.concatenate([row_p, row_p + N]).reshape(-1, BLK)
    cidx = jnp.concatenate([col_p, col_p]).reshape(-1, BLK)

    _sc_scatter, _sc_hist = _sc_kernels()
    zrows = jnp.zeros((ZROWS, HALF), jnp.float32)
    ones_h = jnp.ones((BLK, HB), jnp.float32)
    hist = _sc_hist(ones_h, cidx, zrows)
    deg = hist[0, :N, 0][:, None]

    b0r, b1r, b2r = b0[None, :], b1[None, :], b2[None, :]
    h1, g1, dis = _tc_first(deg, x, W0, b0r)
    acc = _sc_scatter(g1.reshape(2 * N, HALF), ridx, cidx, zrows)
    h2, g2 = _tc_mid(acc, h1, dis, W1, b1r)
    acc = _sc_scatter(g2.reshape(2 * N, HALF), ridx, cidx, zrows)
    h3, g3 = _tc_mid(acc, h2, dis, W2, b2r)
    acc = _sc_scatter(g3.reshape(2 * N, HALF), ridx, cidx, zrows)
    return _tc_final(acc, h3, dis)
